# Initial kernel scaffold; baseline (speedup 1.0000x reference)
import jax
import jax.numpy as jnp
from jax.experimental import pallas as pl


def kernel(x1, adj1, x2, adj2, clm, W11, b11, W12, b12, W21, b21, W22, b22):
    raise NotImplementedError("write your pallas kernel here")


# speedup vs baseline: 1.4666x; 1.4666x over previous
import jax
import jax.numpy as jnp
from jax.experimental import pallas as pl
from jax.experimental.pallas import tpu as pltpu

N = 4096
F = 256
H = 128
TAU = 0.5


def _mm_kernel(x_ref, w_ref, o_ref):
    o_ref[...] = jnp.dot(x_ref[...], w_ref[...],
                         preferred_element_type=jnp.float32)


def _support(x, w, bm=512):
    return pl.pallas_call(
        _mm_kernel,
        grid=(N // bm,),
        in_specs=[
            pl.BlockSpec((bm, F), lambda i: (i, 0)),
            pl.BlockSpec((F, F), lambda i: (0, 0)),
        ],
        out_specs=pl.BlockSpec((bm, F), lambda i: (i, 0)),
        out_shape=jax.ShapeDtypeStruct((N, F), jnp.float32),
    )(x, w)


def _layer1_kernel(adj_ref, sup_ref, b_ref, w2_ref, o_ref):
    acc = jnp.dot(adj_ref[...], sup_ref[...],
                  preferred_element_type=jnp.float32)
    h = jnp.maximum(acc + b_ref[...], 0.0)
    o_ref[...] = jnp.dot(h, w2_ref[...], preferred_element_type=jnp.float32)


def _layer1(adj, sup, b1, w2, bm=256):
    return pl.pallas_call(
        _layer1_kernel,
        grid=(N // bm,),
        in_specs=[
            pl.BlockSpec((bm, N), lambda i: (i, 0)),
            pl.BlockSpec((N, F), lambda i: (0, 0)),
            pl.BlockSpec((1, F), lambda i: (0, 0)),
            pl.BlockSpec((F, H), lambda i: (0, 0)),
        ],
        out_specs=pl.BlockSpec((bm, H), lambda i: (i, 0)),
        out_shape=jax.ShapeDtypeStruct((N, H), jnp.float32),
    )(adj, sup, b1.reshape(1, F), w2)


def _layer2_kernel(adj_ref, s2_ref, b_ref, o_ref):
    acc = jnp.dot(adj_ref[...], s2_ref[...],
                  preferred_element_type=jnp.float32)
    o_ref[...] = acc + b_ref[...]


def _layer2(adj, s2, b2, bm=256):
    return pl.pallas_call(
        _layer2_kernel,
        grid=(N // bm,),
        in_specs=[
            pl.BlockSpec((bm, N), lambda i: (i, 0)),
            pl.BlockSpec((N, H), lambda i: (0, 0)),
            pl.BlockSpec((1, H), lambda i: (0, 0)),
        ],
        out_specs=pl.BlockSpec((bm, H), lambda i: (i, 0)),
        out_shape=jax.ShapeDtypeStruct((N, H), jnp.float32),
    )(adj, s2, b2.reshape(1, H))


def _sim_kernel(z1_ref, z2_ref, clm_ref, loss_ref, acc_ref):
    i = pl.program_id(0)
    z1 = z1_ref[...]
    z2 = z2_ref[...]
    r1 = jax.lax.rsqrt(jnp.sum(z1 * z1, axis=1, keepdims=True)) * (1.0 / TAU)
    r2 = jax.lax.rsqrt(jnp.sum(z2 * z2, axis=1, keepdims=True))
    s = jax.lax.dot_general(z1, z2, (((1,), (1,)), ((), ())),
                            preferred_element_type=jnp.float32)
    p = jnp.exp(s * r1 * r2.reshape(1, -1))
    rs = jnp.sum(p, axis=1, keepdims=True)
    ws = jnp.sum(p * clm_ref[...], axis=1, keepdims=True)
    part = jnp.sum(jnp.log(rs + 1e-8) - jnp.log(ws))

    @pl.when(i == 0)
    def _():
        acc_ref[0] = 0.0

    acc_ref[0] += part

    @pl.when(i == pl.num_programs(0) - 1)
    def _():
        loss_ref[...] = jnp.full((1, 1), acc_ref[0] * (1.0 / N),
                                 dtype=jnp.float32)


def _sim_loss(z1, z2, clm, bm=256):
    loss = pl.pallas_call(
        _sim_kernel,
        grid=(N // bm,),
        in_specs=[
            pl.BlockSpec((bm, H), lambda i: (i, 0)),
            pl.BlockSpec((N, H), lambda i: (0, 0)),
            pl.BlockSpec((bm, N), lambda i: (i, 0)),
        ],
        out_specs=pl.BlockSpec((1, 1), lambda i: (0, 0)),
        out_shape=jax.ShapeDtypeStruct((1, 1), jnp.float32),
        scratch_shapes=[pltpu.SMEM((1,), jnp.float32)],
    )(z1, z2, clm)
    return loss.reshape(())


def kernel(x1, adj1, x2, adj2, clm, W11, b11, W12, b12, W21, b21, W22, b22):
    sup1 = _support(x1, W11)
    s2_1 = _layer1(adj1, sup1, b11, W12)
    z1 = _layer2(adj1, s2_1, b12)

    sup2 = _support(x2, W21)
    s2_2 = _layer1(adj2, sup2, b21, W22)
    z2 = _layer2(adj2, s2_2, b22)

    loss = _sim_loss(z1, z2, clm)
    return (z1, z2, loss)


# speedup vs baseline: 1.7618x; 1.2013x over previous
import jax
import jax.numpy as jnp
from jax.experimental import pallas as pl
from jax.experimental.pallas import tpu as pltpu

N = 4096
F = 256
H = 128
TAU = 0.5


def _support_kernel(x_ref, w_ref, o_ref):
    acc = jnp.dot(x_ref[...], w_ref[...], preferred_element_type=jnp.float32)
    o_ref[...] = acc.astype(jnp.bfloat16)


def _support(x, w, bm=512):
    return pl.pallas_call(
        _support_kernel,
        grid=(N // bm,),
        in_specs=[
            pl.BlockSpec((bm, F), lambda i: (i, 0)),
            pl.BlockSpec((F, F), lambda i: (0, 0)),
        ],
        out_specs=pl.BlockSpec((bm, F), lambda i: (i, 0)),
        out_shape=jax.ShapeDtypeStruct((N, F), jnp.bfloat16),
    )(x, w)


def _tower_kernel(adj_ref, sup_ref, b1_ref, w2_ref, b2_ref, o_ref,
                  adj_scr, s2_scr):
    p = pl.program_id(0)
    i = pl.program_id(1)
    bm = o_ref.shape[0]

    @pl.when(p == 0)
    def _():
        ab = adj_ref[...].astype(jnp.bfloat16)
        adj_scr[pl.ds(i * bm, bm), :] = ab
        acc = jnp.dot(ab, sup_ref[...], preferred_element_type=jnp.float32)
        h = jnp.maximum(acc + b1_ref[...], 0.0)
        s2 = jnp.dot(h.astype(jnp.bfloat16), w2_ref[...],
                     preferred_element_type=jnp.float32)
        s2_scr[pl.ds(i * bm, bm), :] = s2.astype(jnp.bfloat16)

    @pl.when(p == 1)
    def _():
        ab = adj_scr[pl.ds(i * bm, bm), :]
        z = jnp.dot(ab, s2_scr[...], preferred_element_type=jnp.float32)
        o_ref[...] = z + b2_ref[...]


def _tower(adj, sup, b1, w2, b2, bm=256):
    ni = N // bm
    return pl.pallas_call(
        _tower_kernel,
        grid=(2, ni),
        in_specs=[
            pl.BlockSpec((bm, N), lambda p, i: ((1 - p) * i + p * (ni - 1), 0)),
            pl.BlockSpec((N, F), lambda p, i: (0, 0)),
            pl.BlockSpec((1, F), lambda p, i: (0, 0)),
            pl.BlockSpec((F, H), lambda p, i: (0, 0)),
            pl.BlockSpec((1, H), lambda p, i: (0, 0)),
        ],
        out_specs=pl.BlockSpec((bm, H), lambda p, i: (p * i, 0)),
        out_shape=jax.ShapeDtypeStruct((N, H), jnp.float32),
        scratch_shapes=[
            pltpu.VMEM((N, N), jnp.bfloat16),
            pltpu.VMEM((N, H), jnp.bfloat16),
        ],
        compiler_params=pltpu.CompilerParams(
            vmem_limit_bytes=110 * 1024 * 1024,
        ),
    )(adj, sup, b1.reshape(1, F), w2.astype(jnp.bfloat16), b2.reshape(1, H))


def _sim_kernel(z1_ref, z2_ref, clm_ref, loss_ref, acc_ref):
    i = pl.program_id(0)
    z1 = z1_ref[...]
    z2 = z2_ref[...]
    r1 = jax.lax.rsqrt(jnp.sum(z1 * z1, axis=1, keepdims=True)) * (1.0 / TAU)
    r2 = jax.lax.rsqrt(jnp.sum(z2 * z2, axis=1, keepdims=True))
    s = jax.lax.dot_general(z1, z2, (((1,), (1,)), ((), ())),
                            preferred_element_type=jnp.float32)
    p = jnp.exp(s * r1 * r2.reshape(1, -1))
    rs = jnp.sum(p, axis=1, keepdims=True)
    ws = jnp.sum(p * clm_ref[...], axis=1, keepdims=True)
    part = jnp.sum(jnp.log(rs + 1e-8) - jnp.log(ws))

    @pl.when(i == 0)
    def _():
        acc_ref[0] = 0.0

    acc_ref[0] += part

    @pl.when(i == pl.num_programs(0) - 1)
    def _():
        loss_ref[...] = jnp.full((1, 1), acc_ref[0] * (1.0 / N),
                                 dtype=jnp.float32)


def _sim_loss(z1, z2, clm, bm=256):
    loss = pl.pallas_call(
        _sim_kernel,
        grid=(N // bm,),
        in_specs=[
            pl.BlockSpec((bm, H), lambda i: (i, 0)),
            pl.BlockSpec((N, H), lambda i: (0, 0)),
            pl.BlockSpec((bm, N), lambda i: (i, 0)),
        ],
        out_specs=pl.BlockSpec((1, 1), lambda i: (0, 0)),
        out_shape=jax.ShapeDtypeStruct((1, 1), jnp.float32),
        scratch_shapes=[pltpu.SMEM((1,), jnp.float32)],
    )(z1, z2, clm)
    return loss.reshape(())


def kernel(x1, adj1, x2, adj2, clm, W11, b11, W12, b12, W21, b21, W22, b22):
    sup1 = _support(x1, W11)
    z1 = _tower(adj1, sup1, b11, W12, b12)

    sup2 = _support(x2, W21)
    z2 = _tower(adj2, sup2, b21, W22, b22)

    loss = _sim_loss(z1, z2, clm)
    return (z1, z2, loss)


# speedup vs baseline: 1.8771x; 1.0655x over previous
import jax
import jax.numpy as jnp
from jax.experimental import pallas as pl
from jax.experimental.pallas import tpu as pltpu

N = 4096
F = 256
H = 128
TAU = 0.5


def _support_kernel(x_ref, w_ref, o_ref):
    acc = jnp.dot(x_ref[...], w_ref[...], preferred_element_type=jnp.float32)
    o_ref[...] = acc.astype(jnp.bfloat16)


def _support(x, w, bm=512):
    return pl.pallas_call(
        _support_kernel,
        grid=(N // bm,),
        in_specs=[
            pl.BlockSpec((bm, F), lambda i: (i, 0)),
            pl.BlockSpec((F, F), lambda i: (0, 0)),
        ],
        out_specs=pl.BlockSpec((bm, F), lambda i: (i, 0)),
        out_shape=jax.ShapeDtypeStruct((N, F), jnp.bfloat16),
    )(x, w)


def _towers_kernel(adj1_ref, adj2_ref, sup1_ref, sup2_ref,
                   b11_ref, w12_ref, b12_ref, b21_ref, w22_ref, b22_ref,
                   z1_ref, z2_ref, adj_scr, s2a_scr, s2b_scr):
    p = pl.program_id(0)
    i = pl.program_id(1)
    bm = z1_ref.shape[0]

    @pl.when(p == 0)
    def _():
        ab = adj1_ref[...].astype(jnp.bfloat16)
        adj_scr[pl.ds(i * bm, bm), :] = ab
        acc = jnp.dot(ab, sup1_ref[...], preferred_element_type=jnp.float32)
        h = jnp.maximum(acc + b11_ref[...], 0.0)
        s2 = jnp.dot(h.astype(jnp.bfloat16), w12_ref[...],
                     preferred_element_type=jnp.float32)
        s2a_scr[pl.ds(i * bm, bm), :] = s2.astype(jnp.bfloat16)

    @pl.when(p == 1)
    def _():
        a1 = adj_scr[pl.ds(i * bm, bm), :]
        z1_ref[...] = jnp.dot(a1, s2a_scr[...],
                              preferred_element_type=jnp.float32) + b12_ref[...]
        ab = adj2_ref[...].astype(jnp.bfloat16)
        adj_scr[pl.ds(i * bm, bm), :] = ab
        acc = jnp.dot(ab, sup2_ref[...], preferred_element_type=jnp.float32)
        h = jnp.maximum(acc + b21_ref[...], 0.0)
        s2 = jnp.dot(h.astype(jnp.bfloat16), w22_ref[...],
                     preferred_element_type=jnp.float32)
        s2b_scr[pl.ds(i * bm, bm), :] = s2.astype(jnp.bfloat16)

    @pl.when(p == 2)
    def _():
        a2 = adj_scr[pl.ds(i * bm, bm), :]
        z2_ref[...] = jnp.dot(a2, s2b_scr[...],
                              preferred_element_type=jnp.float32) + b22_ref[...]


def _towers(adj1, adj2, sup1, sup2, b11, W12, b12, b21, W22, b22, bm=256):
    ni = N // bm
    z1, z2 = pl.pallas_call(
        _towers_kernel,
        grid=(3, ni),
        in_specs=[
            pl.BlockSpec((bm, N),
                         lambda p, i: (jnp.where(p == 0, i, ni - 1), 0)),
            pl.BlockSpec((bm, N),
                         lambda p, i: (jnp.where(p == 1, i,
                                                 jnp.where(p == 0, 0, ni - 1)),
                                       0)),
            pl.BlockSpec((N, F), lambda p, i: (0, 0)),
            pl.BlockSpec((N, F), lambda p, i: (0, 0)),
            pl.BlockSpec((1, F), lambda p, i: (0, 0)),
            pl.BlockSpec((F, H), lambda p, i: (0, 0)),
            pl.BlockSpec((1, H), lambda p, i: (0, 0)),
            pl.BlockSpec((1, F), lambda p, i: (0, 0)),
            pl.BlockSpec((F, H), lambda p, i: (0, 0)),
            pl.BlockSpec((1, H), lambda p, i: (0, 0)),
        ],
        out_specs=[
            pl.BlockSpec((bm, H),
                         lambda p, i: (jnp.where(p == 0, 0,
                                                 jnp.where(p == 1, i, ni - 1)),
                                       0)),
            pl.BlockSpec((bm, H), lambda p, i: (jnp.where(p == 2, i, 0), 0)),
        ],
        out_shape=[
            jax.ShapeDtypeStruct((N, H), jnp.float32),
            jax.ShapeDtypeStruct((N, H), jnp.float32),
        ],
        scratch_shapes=[
            pltpu.VMEM((N, N), jnp.bfloat16),
            pltpu.VMEM((N, H), jnp.bfloat16),
            pltpu.VMEM((N, H), jnp.bfloat16),
        ],
        compiler_params=pltpu.CompilerParams(
            vmem_limit_bytes=110 * 1024 * 1024,
        ),
    )(adj1, adj2, sup1, sup2,
      b11.reshape(1, F), W12.astype(jnp.bfloat16), b12.reshape(1, H),
      b21.reshape(1, F), W22.astype(jnp.bfloat16), b22.reshape(1, H))
    return z1, z2


def _sim_kernel(z1_ref, z2_ref, clm_ref, loss_ref, acc_ref):
    i = pl.program_id(0)
    z1 = z1_ref[...]
    z2 = z2_ref[...]
    r1 = jax.lax.rsqrt(jnp.sum(z1 * z1, axis=1, keepdims=True)) * (1.0 / TAU)
    r2 = jax.lax.rsqrt(jnp.sum(z2 * z2, axis=1, keepdims=True))
    s = jax.lax.dot_general(z1, z2, (((1,), (1,)), ((), ())),
                            preferred_element_type=jnp.float32)
    p = jnp.exp(s * r1 * r2.reshape(1, -1))
    rs = jnp.sum(p, axis=1, keepdims=True)
    ws = jnp.sum(p * clm_ref[...], axis=1, keepdims=True)
    part = jnp.sum(jnp.log(rs + 1e-8) - jnp.log(ws))

    @pl.when(i == 0)
    def _():
        acc_ref[0] = 0.0

    acc_ref[0] += part

    @pl.when(i == pl.num_programs(0) - 1)
    def _():
        loss_ref[...] = jnp.full((1, 1), acc_ref[0] * (1.0 / N),
                                 dtype=jnp.float32)


def _sim_loss(z1, z2, clm, bm=256):
    loss = pl.pallas_call(
        _sim_kernel,
        grid=(N // bm,),
        in_specs=[
            pl.BlockSpec((bm, H), lambda i: (i, 0)),
            pl.BlockSpec((N, H), lambda i: (0, 0)),
            pl.BlockSpec((bm, N), lambda i: (i, 0)),
        ],
        out_specs=pl.BlockSpec((1, 1), lambda i: (0, 0)),
        out_shape=jax.ShapeDtypeStruct((1, 1), jnp.float32),
        scratch_shapes=[pltpu.SMEM((1,), jnp.float32)],
    )(z1, z2, clm)
    return loss.reshape(())


def kernel(x1, adj1, x2, adj2, clm, W11, b11, W12, b12, W21, b21, W22, b22):
    sup1 = _support(x1, W11)
    sup2 = _support(x2, W21)
    z1, z2 = _towers(adj1, adj2, sup1, sup2, b11, W12, b12, b21, W22, b22)
    loss = _sim_loss(z1, z2, clm)
    return (z1, z2, loss)
